# baseline (speedup 1.0000x reference)
import functools

import jax
import jax.numpy as jnp
from jax import lax
from jax.experimental import pallas as pl
from jax.experimental.pallas import tpu as pltpu
from jax.experimental.pallas import tpu_sc as plsc

N = 10000
E = 320000
DF = 128
DE = 16
H = 128
L = 2
B = 4
NPG = N // B

NC = 2
NS = 16
NW = NC * NS
EPT = E // NW
K = 40
NITER = EPT // K
NPAD = 10240
RPS = NPAD // NS
ZR = 128

_ew = functools.partial(pl.BlockSpec, index_map=lambda i: (0, 0))


def _pack_bf16_halves(v):
    u = lax.bitcast_convert_type(v, jnp.uint32)
    r16 = (u + jnp.uint32(0x7FFF)
           + ((u >> jnp.uint32(16)) & jnp.uint32(1))) >> jnp.uint32(16)
    lo = r16[:, :H // 2]
    hi = r16[:, H // 2:]
    return lo | (hi << jnp.uint32(16))


EB = 2000


def _edge_kernel(ea_ref, w1_ref, b1_ref, w2_ref, b2_ref, cw_ref, clb_ref,
                 el_ref):
    r = jnp.maximum(
        jnp.dot(ea_ref[...], w1_ref[...], preferred_element_type=jnp.float32)
        + b1_ref[...], 0.0)
    e = jnp.dot(r, w2_ref[...],
                preferred_element_type=jnp.float32) + b2_ref[...]
    el = jnp.dot(e, cw_ref[0],
                 preferred_element_type=jnp.float32) + clb_ref[0]
    el_ref[...] = _pack_bf16_halves(el)


def _edge_feats(l, edge_attr, w1, b1, w2, b2, cw, clb):
    return pl.pallas_call(
        _edge_kernel,
        grid=(E // EB,),
        in_specs=[
            pl.BlockSpec((EB, DE), lambda i: (i, 0)),
            _ew((DE, H)), _ew((1, H)),
            _ew((H, H)), _ew((1, H)),
            pl.BlockSpec((1, H, H), lambda i, _l=l: (_l, 0, 0)),
            pl.BlockSpec((1, 1, H), lambda i, _l=l: (_l, 0, 0)),
        ],
        out_specs=pl.BlockSpec((EB, H // 2), lambda i: (i, 0)),
        out_shape=jax.ShapeDtypeStruct((E, H // 2), jnp.uint32),
    )(edge_attr, w1, b1.reshape(1, H), w2, b2.reshape(1, H),
      cw, clb.reshape(L, 1, H))


NB = 1000


def _h0_kernel(x_ref, w_ref, b_ref, h_ref):
    t = (jnp.dot(x_ref[...], w_ref[...], preferred_element_type=jnp.float32)
         + b_ref[...])
    h_ref[...] = t * jax.nn.sigmoid(t)


def _h0(x_in, in_w, in_b):
    return pl.pallas_call(
        _h0_kernel,
        grid=(N // NB,),
        in_specs=[
            pl.BlockSpec((NB, DF + 1), lambda i: (i, 0)),
            _ew((DF + 1, H)), _ew((1, H)),
        ],
        out_specs=pl.BlockSpec((NB, H), lambda i: (i, 0)),
        out_shape=jax.ShapeDtypeStruct((N, H), jnp.float32),
    )(x_in, in_w, in_b.reshape(1, H))


def _sc_body(h_hbm, e_hbm, src_hbm, dst_hbm, out_hbm,
             src_v, dst_v, hrows_v, el_v, msg_v, agg_sh, sem_g, sem_e, sem_i):
    cid = lax.axis_index("c")
    sid = lax.axis_index("s")
    tid = cid * NS + sid

    def zbody(i, _):
        for kk in range(H // 16):
            msg_v[i, pl.ds(kk * 16, 16)] = jnp.zeros((16,), jnp.float32)
        return 0
    lax.fori_loop(0, K, zbody, 0)
    for j in range(RPS // K):
        pltpu.sync_copy(msg_v, agg_sh.at[pl.ds(sid * RPS + j * K, K)])
    plsc.subcore_barrier()

    def issue_idx(ch, p):
        pltpu.async_copy(src_hbm.at[tid, pl.ds(ch, 1)],
                         src_v.at[pl.ds(p, 1)], sem_i.at[p])
        pltpu.async_copy(dst_hbm.at[tid, pl.ds(ch, 1)],
                         dst_v.at[pl.ds(p, 1)], sem_i.at[p])

    def wait_idx(ch, p):
        pltpu.make_async_copy(src_hbm.at[tid, pl.ds(ch, 1)],
                              src_v.at[pl.ds(p, 1)], sem_i.at[p]).wait()
        pltpu.make_async_copy(dst_hbm.at[tid, pl.ds(ch, 1)],
                              dst_v.at[pl.ds(p, 1)], sem_i.at[p]).wait()

    def issue_data(ch, p):
        pltpu.async_copy(h_hbm.at[src_v.at[p]], hrows_v.at[p], sem_g.at[p])
        pltpu.async_copy(e_hbm.at[pl.ds(tid * EPT + ch * K, K)],
                         el_v.at[p], sem_e.at[p])

    def wait_data(ch, p):
        pltpu.make_async_copy(h_hbm.at[src_v.at[p]], hrows_v.at[p],
                              sem_g.at[p]).wait()
        pltpu.make_async_copy(e_hbm.at[pl.ds(tid * EPT + ch * K, K)],
                              el_v.at[p], sem_e.at[p]).wait()

    pltpu.sync_copy(src_hbm.at[tid, pl.ds(0, 1)], src_v.at[pl.ds(0, 1)])
    pltpu.sync_copy(dst_hbm.at[tid, pl.ds(0, 1)], dst_v.at[pl.ds(0, 1)])
    issue_data(0, 0)
    issue_idx(1, 1)

    def compute_scatter(p):
        himask = jnp.full((16,), 0xFFFF0000, jnp.uint32)
        sixteen = jnp.full((16,), 16, jnp.uint32)

        def rbody(j, _):
            for kk in range(H // 32):
                sl = pl.ds(kk * 16, 16)
                slh = pl.ds(H // 2 + kk * 16, 16)
                eu = el_v[p, j, sl]
                e_lo = lax.bitcast_convert_type(eu << sixteen, jnp.float32)
                e_hi = lax.bitcast_convert_type(eu & himask, jnp.float32)
                msg_v[j, sl] = jnp.maximum(e_lo + hrows_v[p, j, sl], 0.0)
                msg_v[j, slh] = jnp.maximum(e_hi + hrows_v[p, j, slh], 0.0)
            return 0
        lax.fori_loop(0, K, rbody, 0)
        pltpu.sync_copy(msg_v, agg_sh.at[dst_v.at[p]], add=True)

    def step(ch, p):
        wait_idx(ch + 1, 1 - p)
        issue_data(ch + 1, 1 - p)
        wait_data(ch, p)
        compute_scatter(p)
        issue_idx(ch + 2, p)

    def body(t, _):
        ch = t * 2
        step(ch, 0)
        step(ch + 1, 1)
        return 0
    lax.fori_loop(0, (NITER - 2) // 2, body, 0)

    c2, c1 = jnp.int32(NITER - 2), jnp.int32(NITER - 1)
    wait_idx(c1, 1)
    issue_data(c1, 1)
    wait_data(c2, 0)
    compute_scatter(0)

    wait_data(c1, 1)
    compute_scatter(1)

    plsc.subcore_barrier()
    for j in range(RPS // ZR):
        rows = pl.ds(sid * RPS + j * ZR, ZR)
        pltpu.sync_copy(agg_sh.at[rows], out_hbm.at[cid, rows])


@functools.lru_cache(maxsize=1)
def _get_sc_agg():
    mesh = plsc.VectorSubcoreMesh(core_axis_name="c", subcore_axis_name="s",
                                  num_cores=NC, num_subcores=NS)
    return pl.kernel(
        _sc_body,
        out_type=jax.ShapeDtypeStruct((NC, NPAD, H), jnp.float32),
        mesh=mesh,
        scratch_types=[
            pltpu.VMEM((2, K), jnp.int32),
            pltpu.VMEM((2, K), jnp.int32),
            pltpu.VMEM((2, K, H), jnp.float32),
            pltpu.VMEM((2, K, H // 2), jnp.uint32),
            pltpu.VMEM((K, H), jnp.float32),
            pltpu.VMEM_SHARED((NPAD, H), jnp.float32),
            pltpu.SemaphoreType.DMA((2,)),
            pltpu.SemaphoreType.DMA((2,)),
            pltpu.SemaphoreType.DMA((2,)),
        ],
    )


def _node_kernel(h_ref, a0_ref, a1_ref, eps_ref, w1_ref, b1_ref, w2_ref,
                 b2_ref, g_ref, bb_ref, out_ref):
    z = (1.0 + eps_ref[0, 0]) * h_ref[...] + a0_ref[0] + a1_ref[0]
    t = jnp.dot(z, w1_ref[...], preferred_element_type=jnp.float32) + b1_ref[...]
    t = t * jax.nn.sigmoid(t)
    hn = jnp.dot(t, w2_ref[...], preferred_element_type=jnp.float32) + b2_ref[...]
    mu = jnp.mean(hn, axis=1, keepdims=True)
    var = jnp.mean((hn - mu) * (hn - mu), axis=1, keepdims=True)
    hn = (hn - mu) / jnp.sqrt(var + 1e-5) * g_ref[...] + bb_ref[...]
    out_ref[...] = h_ref[...] + hn * jax.nn.sigmoid(hn)


def _node_update(h, agg2, eps_l, w1, b1, w2, b2, g, bb):
    return pl.pallas_call(
        _node_kernel,
        grid=(N // NB,),
        in_specs=[
            pl.BlockSpec((NB, H), lambda i: (i, 0)),
            pl.BlockSpec((1, NB, H), lambda i: (0, i, 0)),
            pl.BlockSpec((1, NB, H), lambda i: (1, i, 0)),
            _ew((1, 1)),
            _ew((H, H)), _ew((1, H)),
            _ew((H, H)), _ew((1, H)),
            _ew((1, H)), _ew((1, H)),
        ],
        out_specs=pl.BlockSpec((NB, H), lambda i: (i, 0)),
        out_shape=jax.ShapeDtypeStruct((N, H), jnp.float32),
    )(h, agg2, agg2, eps_l, w1, b1.reshape(1, H), w2, b2.reshape(1, H),
      g.reshape(1, H), bb.reshape(1, H))


def _score_kernel(h_ref, w1_ref, b1_ref, w2_ref, b2_ref, s_ref):
    t = jnp.tanh(
        jnp.dot(h_ref[...], w1_ref[...], preferred_element_type=jnp.float32)
        + b1_ref[...])
    s_ref[...] = jnp.dot(t, w2_ref[...],
                         preferred_element_type=jnp.float32) + b2_ref[...]


def _scores(h, w1, b1, w2, b2):
    return pl.pallas_call(
        _score_kernel,
        grid=(N // NB,),
        in_specs=[
            pl.BlockSpec((NB, H), lambda i: (i, 0)),
            _ew((H, H)), _ew((1, H)), _ew((H, 1)), _ew((1, 1)),
        ],
        out_specs=pl.BlockSpec((NB, 1), lambda i: (i, 0)),
        out_shape=jax.ShapeDtypeStruct((N, 1), jnp.float32),
    )(h, w1, b1.reshape(1, H), w2, b2.reshape(1, 1))


def _pool_kernel(sall_ref, h_ref, w1_ref, b1_ref, w2_ref, b2_ref, q_ref):
    sall = sall_ref[...]
    mx = jnp.max(sall)
    u = jnp.exp(sall - mx)
    ssum = jnp.sum(u)
    segexp = jnp.sum(u, axis=1, keepdims=True)
    vs = [jnp.dot(u[bb:bb + 1], h_ref[bb],
                  preferred_element_type=jnp.float32,
                  precision=lax.Precision.HIGHEST) for bb in range(B)]
    v = jnp.concatenate(vs, axis=0)
    pooled = v / (ssum * (segexp / ssum + 1e-8))
    t = jnp.dot(pooled, w1_ref[...],
                preferred_element_type=jnp.float32) + b1_ref[...]
    t = t * jax.nn.sigmoid(t)
    q_ref[...] = jnp.dot(t, w2_ref[...],
                         preferred_element_type=jnp.float32) + b2_ref[...]


def _pool(s_r, h3, w1, b1, w2, b2):
    return pl.pallas_call(
        _pool_kernel,
        in_specs=[
            pl.BlockSpec((B, NPG), lambda: (0, 0)),
            pl.BlockSpec((B, NPG, H), lambda: (0, 0, 0)),
            pl.BlockSpec((H, H), lambda: (0, 0)),
            pl.BlockSpec((1, H), lambda: (0, 0)),
            pl.BlockSpec((H, 1), lambda: (0, 0)),
            pl.BlockSpec((1, 1), lambda: (0, 0)),
        ],
        out_specs=pl.BlockSpec((B, 1), lambda: (0, 0)),
        out_shape=jax.ShapeDtypeStruct((B, 1), jnp.float32),
    )(s_r, h3, w1, b1.reshape(1, H), w2, b2.reshape(1, 1))


def kernel(x, edge_index, edge_attr, batch, ptr, c, edge_W1, edge_b1,
           edge_W2, edge_b2, in_W, in_b, conv_lin_W, conv_lin_b, conv_W1,
           conv_b1, conv_W2, conv_b2, eps, ln_g, ln_b, attn_W1, attn_b1,
           attn_W2, attn_b2, head_W1, head_b1, head_W2, head_b2):
    e0 = _edge_feats(0, edge_attr, edge_W1, edge_b1, edge_W2, edge_b2,
                     conv_lin_W, conv_lin_b)
    x_in = jnp.concatenate([x, c.reshape(N, 1)], axis=1)
    h = _h0(x_in, in_W, in_b)
    e1 = _edge_feats(1, edge_attr, edge_W1, edge_b1, edge_W2, edge_b2,
                     conv_lin_W, conv_lin_b)

    srcr = edge_index[0].reshape(NW, NITER, K)
    dstr = edge_index[1].reshape(NW, NITER, K)
    els = (e0, e1)
    sc_agg = _get_sc_agg()
    for l in range(L):
        agg2 = sc_agg(h, els[l], srcr, dstr)
        h = _node_update(h, agg2, eps[l].reshape(1, 1), conv_W1[l],
                         conv_b1[l], conv_W2[l], conv_b2[l], ln_g[l],
                         ln_b[l])

    s = _scores(h, attn_W1, attn_b1, attn_W2, attn_b2)
    q = _pool(s.reshape(B, NPG), h.reshape(B, NPG, H), head_W1, head_b1,
              head_W2, head_b2)
    return q[:, 0]


# speedup vs baseline: 1.1500x; 1.1500x over previous
import functools

import jax
import jax.numpy as jnp
from jax import lax
from jax.experimental import pallas as pl
from jax.experimental.pallas import tpu as pltpu
from jax.experimental.pallas import tpu_sc as plsc

N = 10000
E = 320000
DF = 128
DE = 16
H = 128
L = 2
B = 4
NPG = N // B

NC = 2
NS = 16
NW = NC * NS
EPT = E // NW
K = 80
KH = K // 2
NITER = EPT // K
NPAD = 10240
RPS = NPAD // NS
ZR = 128

_ew = functools.partial(pl.BlockSpec, index_map=lambda i: (0, 0))


def _pack_bf16_halves(v):
    u = lax.bitcast_convert_type(v, jnp.uint32)
    r16 = (u + jnp.uint32(0x7FFF)
           + ((u >> jnp.uint32(16)) & jnp.uint32(1))) >> jnp.uint32(16)
    lo = r16[:, :H // 2]
    hi = r16[:, H // 2:]
    return lo | (hi << jnp.uint32(16))


EB = 2000


def _edge_kernel(ea_ref, w1_ref, b1_ref, w2_ref, b2_ref, cw_ref, clb_ref,
                 el_ref):
    r = jnp.maximum(
        jnp.dot(ea_ref[...], w1_ref[...], preferred_element_type=jnp.float32)
        + b1_ref[...], 0.0)
    e = jnp.dot(r, w2_ref[...],
                preferred_element_type=jnp.float32) + b2_ref[...]
    el = jnp.dot(e, cw_ref[0],
                 preferred_element_type=jnp.float32) + clb_ref[0]
    el_ref[...] = _pack_bf16_halves(el)


def _edge_feats(l, edge_attr, w1, b1, w2, b2, cw, clb):
    return pl.pallas_call(
        _edge_kernel,
        grid=(E // EB,),
        in_specs=[
            pl.BlockSpec((EB, DE), lambda i: (i, 0)),
            _ew((DE, H)), _ew((1, H)),
            _ew((H, H)), _ew((1, H)),
            pl.BlockSpec((1, H, H), lambda i, _l=l: (_l, 0, 0)),
            pl.BlockSpec((1, 1, H), lambda i, _l=l: (_l, 0, 0)),
        ],
        out_specs=pl.BlockSpec((EB, H // 2), lambda i: (i, 0)),
        out_shape=jax.ShapeDtypeStruct((E, H // 2), jnp.uint32),
    )(edge_attr, w1, b1.reshape(1, H), w2, b2.reshape(1, H),
      cw, clb.reshape(L, 1, H))


NB = 1000


def _h0_kernel(x_ref, w_ref, b_ref, h_ref):
    t = (jnp.dot(x_ref[...], w_ref[...], preferred_element_type=jnp.float32)
         + b_ref[...])
    h_ref[...] = t * jax.nn.sigmoid(t)


def _h0(x_in, in_w, in_b):
    return pl.pallas_call(
        _h0_kernel,
        grid=(N // NB,),
        in_specs=[
            pl.BlockSpec((NB, DF + 1), lambda i: (i, 0)),
            _ew((DF + 1, H)), _ew((1, H)),
        ],
        out_specs=pl.BlockSpec((NB, H), lambda i: (i, 0)),
        out_shape=jax.ShapeDtypeStruct((N, H), jnp.float32),
    )(x_in, in_w, in_b.reshape(1, H))


def _sc_body(h_hbm, e_hbm, src_hbm, dst_hbm, out_hbm,
             src_v, dst_v, hrows_v, el_v, msg_v, agg_sh, sem_g, sem_e, sem_i):
    cid = lax.axis_index("c")
    sid = lax.axis_index("s")
    tid = cid * NS + sid

    def zbody(i, _):
        for kk in range(H // 16):
            msg_v[i, pl.ds(kk * 16, 16)] = jnp.zeros((16,), jnp.float32)
        return 0
    lax.fori_loop(0, KH, zbody, 0)
    for j in range(RPS // KH):
        pltpu.sync_copy(msg_v, agg_sh.at[pl.ds(sid * RPS + j * KH, KH)])
    plsc.subcore_barrier()

    def issue_idx(ch, p):
        pltpu.async_copy(src_hbm.at[tid, pl.ds(ch, 1)],
                         src_v.at[pl.ds(p, 1)], sem_i.at[p])
        pltpu.async_copy(dst_hbm.at[tid, pl.ds(ch, 1)],
                         dst_v.at[pl.ds(p, 1)], sem_i.at[p])

    def wait_idx(ch, p):
        pltpu.make_async_copy(src_hbm.at[tid, pl.ds(ch, 1)],
                              src_v.at[pl.ds(p, 1)], sem_i.at[p]).wait()
        pltpu.make_async_copy(dst_hbm.at[tid, pl.ds(ch, 1)],
                              dst_v.at[pl.ds(p, 1)], sem_i.at[p]).wait()

    def issue_data(ch, p):
        pltpu.async_copy(h_hbm.at[src_v.at[p]], hrows_v.at[p], sem_g.at[p])
        pltpu.async_copy(e_hbm.at[pl.ds(tid * EPT + ch * K, K)],
                         el_v.at[p], sem_e.at[p])

    def wait_data(ch, p):
        pltpu.make_async_copy(h_hbm.at[src_v.at[p]], hrows_v.at[p],
                              sem_g.at[p]).wait()
        pltpu.make_async_copy(e_hbm.at[pl.ds(tid * EPT + ch * K, K)],
                              el_v.at[p], sem_e.at[p]).wait()

    pltpu.sync_copy(src_hbm.at[tid, pl.ds(0, 1)], src_v.at[pl.ds(0, 1)])
    pltpu.sync_copy(dst_hbm.at[tid, pl.ds(0, 1)], dst_v.at[pl.ds(0, 1)])
    issue_data(0, 0)
    issue_idx(1, 1)

    def compute_scatter(p):
        himask = jnp.full((16,), 0xFFFF0000, jnp.uint32)
        sixteen = jnp.full((16,), 16, jnp.uint32)

        for hh in range(2):
            def rbody(j, _, _hh=hh):
                jr = _hh * KH + j
                for kk in range(H // 32):
                    sl = pl.ds(kk * 16, 16)
                    slh = pl.ds(H // 2 + kk * 16, 16)
                    eu = el_v[p, jr, sl]
                    e_lo = lax.bitcast_convert_type(eu << sixteen, jnp.float32)
                    e_hi = lax.bitcast_convert_type(eu & himask, jnp.float32)
                    msg_v[j, sl] = jnp.maximum(e_lo + hrows_v[p, jr, sl], 0.0)
                    msg_v[j, slh] = jnp.maximum(
                        e_hi + hrows_v[p, jr, slh], 0.0)
                return 0
            lax.fori_loop(0, KH, rbody, 0)
            pltpu.sync_copy(msg_v, agg_sh.at[dst_v.at[p, hh]], add=True)

    def step(ch, p):
        wait_idx(ch + 1, 1 - p)
        issue_data(ch + 1, 1 - p)
        wait_data(ch, p)
        compute_scatter(p)
        issue_idx(ch + 2, p)

    def body(t, _):
        ch = t * 2
        step(ch, 0)
        step(ch + 1, 1)
        return 0
    lax.fori_loop(0, (NITER - 3) // 2, body, 0)

    c3, c2, c1 = (jnp.int32(NITER - 3), jnp.int32(NITER - 2),
                  jnp.int32(NITER - 1))
    wait_idx(c2, 1)
    issue_data(c2, 1)
    wait_data(c3, 0)
    compute_scatter(0)
    issue_idx(c1, 0)

    wait_idx(c1, 0)
    issue_data(c1, 0)
    wait_data(c2, 1)
    compute_scatter(1)

    wait_data(c1, 0)
    compute_scatter(0)

    plsc.subcore_barrier()
    for j in range(RPS // ZR):
        rows = pl.ds(sid * RPS + j * ZR, ZR)
        pltpu.sync_copy(agg_sh.at[rows], out_hbm.at[cid, rows])


@functools.lru_cache(maxsize=1)
def _get_sc_agg():
    mesh = plsc.VectorSubcoreMesh(core_axis_name="c", subcore_axis_name="s",
                                  num_cores=NC, num_subcores=NS)
    return pl.kernel(
        _sc_body,
        out_type=jax.ShapeDtypeStruct((NC, NPAD, H), jnp.float32),
        mesh=mesh,
        scratch_types=[
            pltpu.VMEM((2, K), jnp.int32),
            pltpu.VMEM((2, 2, KH), jnp.int32),
            pltpu.VMEM((2, K, H), jnp.float32),
            pltpu.VMEM((2, K, H // 2), jnp.uint32),
            pltpu.VMEM((KH, H), jnp.float32),
            pltpu.VMEM_SHARED((NPAD, H), jnp.float32),
            pltpu.SemaphoreType.DMA((2,)),
            pltpu.SemaphoreType.DMA((2,)),
            pltpu.SemaphoreType.DMA((2,)),
        ],
    )


def _node_kernel(h_ref, a0_ref, a1_ref, eps_ref, w1_ref, b1_ref, w2_ref,
                 b2_ref, g_ref, bb_ref, out_ref):
    z = (1.0 + eps_ref[0, 0]) * h_ref[...] + a0_ref[0] + a1_ref[0]
    t = jnp.dot(z, w1_ref[...], preferred_element_type=jnp.float32) + b1_ref[...]
    t = t * jax.nn.sigmoid(t)
    hn = jnp.dot(t, w2_ref[...], preferred_element_type=jnp.float32) + b2_ref[...]
    mu = jnp.mean(hn, axis=1, keepdims=True)
    var = jnp.mean((hn - mu) * (hn - mu), axis=1, keepdims=True)
    hn = (hn - mu) / jnp.sqrt(var + 1e-5) * g_ref[...] + bb_ref[...]
    out_ref[...] = h_ref[...] + hn * jax.nn.sigmoid(hn)


def _node_update(h, agg2, eps_l, w1, b1, w2, b2, g, bb):
    return pl.pallas_call(
        _node_kernel,
        grid=(N // NB,),
        in_specs=[
            pl.BlockSpec((NB, H), lambda i: (i, 0)),
            pl.BlockSpec((1, NB, H), lambda i: (0, i, 0)),
            pl.BlockSpec((1, NB, H), lambda i: (1, i, 0)),
            _ew((1, 1)),
            _ew((H, H)), _ew((1, H)),
            _ew((H, H)), _ew((1, H)),
            _ew((1, H)), _ew((1, H)),
        ],
        out_specs=pl.BlockSpec((NB, H), lambda i: (i, 0)),
        out_shape=jax.ShapeDtypeStruct((N, H), jnp.float32),
    )(h, agg2, agg2, eps_l, w1, b1.reshape(1, H), w2, b2.reshape(1, H),
      g.reshape(1, H), bb.reshape(1, H))


def _score_kernel(h_ref, w1_ref, b1_ref, w2_ref, b2_ref, s_ref):
    t = jnp.tanh(
        jnp.dot(h_ref[...], w1_ref[...], preferred_element_type=jnp.float32)
        + b1_ref[...])
    s_ref[...] = jnp.dot(t, w2_ref[...],
                         preferred_element_type=jnp.float32) + b2_ref[...]


def _scores(h, w1, b1, w2, b2):
    return pl.pallas_call(
        _score_kernel,
        grid=(N // NB,),
        in_specs=[
            pl.BlockSpec((NB, H), lambda i: (i, 0)),
            _ew((H, H)), _ew((1, H)), _ew((H, 1)), _ew((1, 1)),
        ],
        out_specs=pl.BlockSpec((NB, 1), lambda i: (i, 0)),
        out_shape=jax.ShapeDtypeStruct((N, 1), jnp.float32),
    )(h, w1, b1.reshape(1, H), w2, b2.reshape(1, 1))


def _pool_kernel(sall_ref, h_ref, w1_ref, b1_ref, w2_ref, b2_ref, q_ref):
    sall = sall_ref[...]
    mx = jnp.max(sall)
    u = jnp.exp(sall - mx)
    ssum = jnp.sum(u)
    segexp = jnp.sum(u, axis=1, keepdims=True)
    vs = [jnp.dot(u[bb:bb + 1], h_ref[bb],
                  preferred_element_type=jnp.float32,
                  precision=lax.Precision.HIGHEST) for bb in range(B)]
    v = jnp.concatenate(vs, axis=0)
    pooled = v / (ssum * (segexp / ssum + 1e-8))
    t = jnp.dot(pooled, w1_ref[...],
                preferred_element_type=jnp.float32) + b1_ref[...]
    t = t * jax.nn.sigmoid(t)
    q_ref[...] = jnp.dot(t, w2_ref[...],
                         preferred_element_type=jnp.float32) + b2_ref[...]


def _pool(s_r, h3, w1, b1, w2, b2):
    return pl.pallas_call(
        _pool_kernel,
        in_specs=[
            pl.BlockSpec((B, NPG), lambda: (0, 0)),
            pl.BlockSpec((B, NPG, H), lambda: (0, 0, 0)),
            pl.BlockSpec((H, H), lambda: (0, 0)),
            pl.BlockSpec((1, H), lambda: (0, 0)),
            pl.BlockSpec((H, 1), lambda: (0, 0)),
            pl.BlockSpec((1, 1), lambda: (0, 0)),
        ],
        out_specs=pl.BlockSpec((B, 1), lambda: (0, 0)),
        out_shape=jax.ShapeDtypeStruct((B, 1), jnp.float32),
    )(s_r, h3, w1, b1.reshape(1, H), w2, b2.reshape(1, 1))


def kernel(x, edge_index, edge_attr, batch, ptr, c, edge_W1, edge_b1,
           edge_W2, edge_b2, in_W, in_b, conv_lin_W, conv_lin_b, conv_W1,
           conv_b1, conv_W2, conv_b2, eps, ln_g, ln_b, attn_W1, attn_b1,
           attn_W2, attn_b2, head_W1, head_b1, head_W2, head_b2):
    e0 = _edge_feats(0, edge_attr, edge_W1, edge_b1, edge_W2, edge_b2,
                     conv_lin_W, conv_lin_b)
    x_in = jnp.concatenate([x, c.reshape(N, 1)], axis=1)
    h = _h0(x_in, in_W, in_b)
    e1 = _edge_feats(1, edge_attr, edge_W1, edge_b1, edge_W2, edge_b2,
                     conv_lin_W, conv_lin_b)

    srcr = edge_index[0].reshape(NW, NITER, K)
    dstr = edge_index[1].reshape(NW, NITER, 2, KH)
    els = (e0, e1)
    sc_agg = _get_sc_agg()
    for l in range(L):
        agg2 = sc_agg(h, els[l], srcr, dstr)
        h = _node_update(h, agg2, eps[l].reshape(1, 1), conv_W1[l],
                         conv_b1[l], conv_W2[l], conv_b2[l], ln_g[l],
                         ln_b[l])

    s = _scores(h, attn_W1, attn_b1, attn_W2, attn_b2)
    q = _pool(s.reshape(B, NPG), h.reshape(B, NPG, H), head_W1, head_b1,
              head_W2, head_b2)
    return q[:, 0]


# speedup vs baseline: 1.1860x; 1.0313x over previous
import functools

import jax
import jax.numpy as jnp
from jax import lax
from jax.experimental import pallas as pl
from jax.experimental.pallas import tpu as pltpu
from jax.experimental.pallas import tpu_sc as plsc

N = 10000
E = 320000
DF = 128
DE = 16
H = 128
L = 2
B = 4
NPG = N // B

NC = 2
NS = 16
NW = NC * NS
EPT = E // NW
K = 80
KH = K // 2
NITER = EPT // K
NPAD = 10240
RPS = NPAD // NS
ZR = 128

_ew = functools.partial(pl.BlockSpec, index_map=lambda i: (0, 0))


def _pack_bf16_halves(v):
    u = lax.bitcast_convert_type(v, jnp.uint32)
    r16 = (u + jnp.uint32(0x7FFF)
           + ((u >> jnp.uint32(16)) & jnp.uint32(1))) >> jnp.uint32(16)
    lo = r16[:, :H // 2]
    hi = r16[:, H // 2:]
    return lo | (hi << jnp.uint32(16))


EB = 2000


def _edge_kernel(ea_ref, w1_ref, b1_ref, w2_ref, b2_ref, cw_ref, clb_ref,
                 el_ref):
    r = jnp.maximum(
        jnp.dot(ea_ref[...], w1_ref[...], preferred_element_type=jnp.float32)
        + b1_ref[...], 0.0)
    e = jnp.dot(r, w2_ref[...],
                preferred_element_type=jnp.float32) + b2_ref[...]
    el = jnp.dot(e, cw_ref[0],
                 preferred_element_type=jnp.float32) + clb_ref[0]
    el_ref[...] = _pack_bf16_halves(el)


def _edge_feats(l, edge_attr, w1, b1, w2, b2, cw, clb):
    return pl.pallas_call(
        _edge_kernel,
        grid=(E // EB,),
        in_specs=[
            pl.BlockSpec((EB, DE), lambda i: (i, 0)),
            _ew((DE, H)), _ew((1, H)),
            _ew((H, H)), _ew((1, H)),
            pl.BlockSpec((1, H, H), lambda i, _l=l: (_l, 0, 0)),
            pl.BlockSpec((1, 1, H), lambda i, _l=l: (_l, 0, 0)),
        ],
        out_specs=pl.BlockSpec((EB, H // 2), lambda i: (i, 0)),
        out_shape=jax.ShapeDtypeStruct((E, H // 2), jnp.uint32),
    )(edge_attr, w1, b1.reshape(1, H), w2, b2.reshape(1, H),
      cw, clb.reshape(L, 1, H))


NB = 1000


def _h0_kernel(x_ref, w_ref, b_ref, h_ref):
    t = (jnp.dot(x_ref[...], w_ref[...], preferred_element_type=jnp.float32)
         + b_ref[...])
    h_ref[...] = t * jax.nn.sigmoid(t)


def _h0(x_in, in_w, in_b):
    return pl.pallas_call(
        _h0_kernel,
        grid=(N // NB,),
        in_specs=[
            pl.BlockSpec((NB, DF + 1), lambda i: (i, 0)),
            _ew((DF + 1, H)), _ew((1, H)),
        ],
        out_specs=pl.BlockSpec((NB, H), lambda i: (i, 0)),
        out_shape=jax.ShapeDtypeStruct((N, H), jnp.float32),
    )(x_in, in_w, in_b.reshape(1, H))


def _sc_body(h_hbm, e_hbm, src_hbm, dst_hbm, out_hbm,
             src_v, dst_v, hrows_v, el_v, agg_sh, sem_g, sem_e, sem_i):
    cid = lax.axis_index("c")
    sid = lax.axis_index("s")
    tid = cid * NS + sid

    def zbody(i, _):
        for kk in range(H // 16):
            hrows_v[0, i, pl.ds(kk * 16, 16)] = jnp.zeros((16,), jnp.float32)
        return 0
    lax.fori_loop(0, K, zbody, 0)
    for j in range(RPS // K):
        pltpu.sync_copy(hrows_v.at[0], agg_sh.at[pl.ds(sid * RPS + j * K, K)])
    plsc.subcore_barrier()

    def issue_idx(ch, p):
        pltpu.async_copy(src_hbm.at[tid, pl.ds(ch, 1)],
                         src_v.at[pl.ds(p, 1)], sem_i.at[p])
        pltpu.async_copy(dst_hbm.at[tid, pl.ds(ch, 1)],
                         dst_v.at[pl.ds(p, 1)], sem_i.at[p])

    def wait_idx(ch, p):
        pltpu.make_async_copy(src_hbm.at[tid, pl.ds(ch, 1)],
                              src_v.at[pl.ds(p, 1)], sem_i.at[p]).wait()
        pltpu.make_async_copy(dst_hbm.at[tid, pl.ds(ch, 1)],
                              dst_v.at[pl.ds(p, 1)], sem_i.at[p]).wait()

    def issue_data(ch, p):
        pltpu.async_copy(h_hbm.at[src_v.at[p]], hrows_v.at[p], sem_g.at[p])
        pltpu.async_copy(e_hbm.at[pl.ds(tid * EPT + ch * K, K)],
                         el_v.at[p], sem_e.at[p])

    def wait_data(ch, p):
        pltpu.make_async_copy(h_hbm.at[src_v.at[p]], hrows_v.at[p],
                              sem_g.at[p]).wait()
        pltpu.make_async_copy(e_hbm.at[pl.ds(tid * EPT + ch * K, K)],
                              el_v.at[p], sem_e.at[p]).wait()

    pltpu.sync_copy(src_hbm.at[tid, pl.ds(0, 1)], src_v.at[pl.ds(0, 1)])
    pltpu.sync_copy(dst_hbm.at[tid, pl.ds(0, 1)], dst_v.at[pl.ds(0, 1)])
    issue_data(0, 0)
    issue_idx(1, 1)

    def compute_scatter(p):
        himask = jnp.full((16,), 0xFFFF0000, jnp.uint32)
        sixteen = jnp.full((16,), 16, jnp.uint32)

        def rbody(j, _):
            for kk in range(H // 32):
                sl = pl.ds(kk * 16, 16)
                slh = pl.ds(H // 2 + kk * 16, 16)
                eu = el_v[p, j, sl]
                e_lo = lax.bitcast_convert_type(eu << sixteen, jnp.float32)
                e_hi = lax.bitcast_convert_type(eu & himask, jnp.float32)
                hrows_v[p, j, sl] = jnp.maximum(
                    e_lo + hrows_v[p, j, sl], 0.0)
                hrows_v[p, j, slh] = jnp.maximum(
                    e_hi + hrows_v[p, j, slh], 0.0)
            return 0
        lax.fori_loop(0, K, rbody, 0)
        pltpu.sync_copy(hrows_v.at[p], agg_sh.at[dst_v.at[p]], add=True)

    def step(ch, p):
        wait_idx(ch + 1, 1 - p)
        issue_data(ch + 1, 1 - p)
        wait_data(ch, p)
        compute_scatter(p)
        issue_idx(ch + 2, p)

    def body(t, _):
        ch = t * 2
        step(ch, 0)
        step(ch + 1, 1)
        return 0
    lax.fori_loop(0, (NITER - 3) // 2, body, 0)

    c3, c2, c1 = (jnp.int32(NITER - 3), jnp.int32(NITER - 2),
                  jnp.int32(NITER - 1))
    wait_idx(c2, 1)
    issue_data(c2, 1)
    wait_data(c3, 0)
    compute_scatter(0)
    issue_idx(c1, 0)

    wait_idx(c1, 0)
    issue_data(c1, 0)
    wait_data(c2, 1)
    compute_scatter(1)

    wait_data(c1, 0)
    compute_scatter(0)

    plsc.subcore_barrier()
    for j in range(RPS // ZR):
        rows = pl.ds(sid * RPS + j * ZR, ZR)
        pltpu.sync_copy(agg_sh.at[rows], out_hbm.at[cid, rows])


@functools.lru_cache(maxsize=1)
def _get_sc_agg():
    mesh = plsc.VectorSubcoreMesh(core_axis_name="c", subcore_axis_name="s",
                                  num_cores=NC, num_subcores=NS)
    return pl.kernel(
        _sc_body,
        out_type=jax.ShapeDtypeStruct((NC, NPAD, H), jnp.float32),
        mesh=mesh,
        scratch_types=[
            pltpu.VMEM((2, K), jnp.int32),
            pltpu.VMEM((2, K), jnp.int32),
            pltpu.VMEM((2, K, H), jnp.float32),
            pltpu.VMEM((2, K, H // 2), jnp.uint32),
            pltpu.VMEM_SHARED((NPAD, H), jnp.float32),
            pltpu.SemaphoreType.DMA((2,)),
            pltpu.SemaphoreType.DMA((2,)),
            pltpu.SemaphoreType.DMA((2,)),
        ],
    )


def _node_kernel(h_ref, a0_ref, a1_ref, eps_ref, w1_ref, b1_ref, w2_ref,
                 b2_ref, g_ref, bb_ref, out_ref):
    z = (1.0 + eps_ref[0, 0]) * h_ref[...] + a0_ref[0] + a1_ref[0]
    t = jnp.dot(z, w1_ref[...], preferred_element_type=jnp.float32) + b1_ref[...]
    t = t * jax.nn.sigmoid(t)
    hn = jnp.dot(t, w2_ref[...], preferred_element_type=jnp.float32) + b2_ref[...]
    mu = jnp.mean(hn, axis=1, keepdims=True)
    var = jnp.mean((hn - mu) * (hn - mu), axis=1, keepdims=True)
    hn = (hn - mu) / jnp.sqrt(var + 1e-5) * g_ref[...] + bb_ref[...]
    out_ref[...] = h_ref[...] + hn * jax.nn.sigmoid(hn)


def _node_update(h, agg2, eps_l, w1, b1, w2, b2, g, bb):
    return pl.pallas_call(
        _node_kernel,
        grid=(N // NB,),
        in_specs=[
            pl.BlockSpec((NB, H), lambda i: (i, 0)),
            pl.BlockSpec((1, NB, H), lambda i: (0, i, 0)),
            pl.BlockSpec((1, NB, H), lambda i: (1, i, 0)),
            _ew((1, 1)),
            _ew((H, H)), _ew((1, H)),
            _ew((H, H)), _ew((1, H)),
            _ew((1, H)), _ew((1, H)),
        ],
        out_specs=pl.BlockSpec((NB, H), lambda i: (i, 0)),
        out_shape=jax.ShapeDtypeStruct((N, H), jnp.float32),
    )(h, agg2, agg2, eps_l, w1, b1.reshape(1, H), w2, b2.reshape(1, H),
      g.reshape(1, H), bb.reshape(1, H))


def _score_kernel(h_ref, w1_ref, b1_ref, w2_ref, b2_ref, s_ref):
    t = jnp.tanh(
        jnp.dot(h_ref[...], w1_ref[...], preferred_element_type=jnp.float32)
        + b1_ref[...])
    s_ref[...] = jnp.dot(t, w2_ref[...],
                         preferred_element_type=jnp.float32) + b2_ref[...]


def _scores(h, w1, b1, w2, b2):
    return pl.pallas_call(
        _score_kernel,
        grid=(N // NB,),
        in_specs=[
            pl.BlockSpec((NB, H), lambda i: (i, 0)),
            _ew((H, H)), _ew((1, H)), _ew((H, 1)), _ew((1, 1)),
        ],
        out_specs=pl.BlockSpec((NB, 1), lambda i: (i, 0)),
        out_shape=jax.ShapeDtypeStruct((N, 1), jnp.float32),
    )(h, w1, b1.reshape(1, H), w2, b2.reshape(1, 1))


def _pool_kernel(sall_ref, h_ref, w1_ref, b1_ref, w2_ref, b2_ref, q_ref):
    sall = sall_ref[...]
    mx = jnp.max(sall)
    u = jnp.exp(sall - mx)
    ssum = jnp.sum(u)
    segexp = jnp.sum(u, axis=1, keepdims=True)
    vs = [jnp.dot(u[bb:bb + 1], h_ref[bb],
                  preferred_element_type=jnp.float32,
                  precision=lax.Precision.HIGHEST) for bb in range(B)]
    v = jnp.concatenate(vs, axis=0)
    pooled = v / (ssum * (segexp / ssum + 1e-8))
    t = jnp.dot(pooled, w1_ref[...],
                preferred_element_type=jnp.float32) + b1_ref[...]
    t = t * jax.nn.sigmoid(t)
    q_ref[...] = jnp.dot(t, w2_ref[...],
                         preferred_element_type=jnp.float32) + b2_ref[...]


def _pool(s_r, h3, w1, b1, w2, b2):
    return pl.pallas_call(
        _pool_kernel,
        in_specs=[
            pl.BlockSpec((B, NPG), lambda: (0, 0)),
            pl.BlockSpec((B, NPG, H), lambda: (0, 0, 0)),
            pl.BlockSpec((H, H), lambda: (0, 0)),
            pl.BlockSpec((1, H), lambda: (0, 0)),
            pl.BlockSpec((H, 1), lambda: (0, 0)),
            pl.BlockSpec((1, 1), lambda: (0, 0)),
        ],
        out_specs=pl.BlockSpec((B, 1), lambda: (0, 0)),
        out_shape=jax.ShapeDtypeStruct((B, 1), jnp.float32),
    )(s_r, h3, w1, b1.reshape(1, H), w2, b2.reshape(1, 1))


def kernel(x, edge_index, edge_attr, batch, ptr, c, edge_W1, edge_b1,
           edge_W2, edge_b2, in_W, in_b, conv_lin_W, conv_lin_b, conv_W1,
           conv_b1, conv_W2, conv_b2, eps, ln_g, ln_b, attn_W1, attn_b1,
           attn_W2, attn_b2, head_W1, head_b1, head_W2, head_b2):
    e0 = _edge_feats(0, edge_attr, edge_W1, edge_b1, edge_W2, edge_b2,
                     conv_lin_W, conv_lin_b)
    x_in = jnp.concatenate([x, c.reshape(N, 1)], axis=1)
    h = _h0(x_in, in_W, in_b)
    e1 = _edge_feats(1, edge_attr, edge_W1, edge_b1, edge_W2, edge_b2,
                     conv_lin_W, conv_lin_b)

    srcr = edge_index[0].reshape(NW, NITER, K)
    dstr = edge_index[1].reshape(NW, NITER, K)
    els = (e0, e1)
    sc_agg = _get_sc_agg()
    for l in range(L):
        agg2 = sc_agg(h, els[l], srcr, dstr)
        h = _node_update(h, agg2, eps[l].reshape(1, 1), conv_W1[l],
                         conv_b1[l], conv_W2[l], conv_b2[l], ln_g[l],
                         ln_b[l])

    s = _scores(h, attn_W1, attn_b1, attn_W2, attn_b2)
    q = _pool(s.reshape(B, NPG), h.reshape(B, NPG, H), head_W1, head_b1,
              head_W2, head_b2)
    return q[:, 0]


# speedup vs baseline: 1.2447x; 1.0495x over previous
import functools

import jax
import jax.numpy as jnp
from jax import lax
from jax.experimental import pallas as pl
from jax.experimental.pallas import tpu as pltpu
from jax.experimental.pallas import tpu_sc as plsc

N = 10000
E = 320000
DF = 128
DE = 16
H = 128
L = 2
B = 4
NPG = N // B

NC = 2
NS = 16
NW = NC * NS
EPT = E // NW
K = 80
KH = K // 2
NITER = EPT // K
NPAD = 10240
RPS = NPAD // NS
ZR = 128

_ew = functools.partial(pl.BlockSpec, index_map=lambda i: (0, 0))


def _pack_bf16_halves(v):
    u = lax.bitcast_convert_type(v, jnp.uint32)
    r16 = (u + jnp.uint32(0x7FFF)
           + ((u >> jnp.uint32(16)) & jnp.uint32(1))) >> jnp.uint32(16)
    lo = r16[:, :H // 2]
    hi = r16[:, H // 2:]
    return lo | (hi << jnp.uint32(16))


EB = 4000


def _edge_kernel(ea_ref, w1_ref, b1_ref, w2_ref, b2_ref, cw_ref, clb_ref,
                 el_ref):
    r = jnp.maximum(
        jnp.dot(ea_ref[...], w1_ref[...], preferred_element_type=jnp.float32)
        + b1_ref[...], 0.0)
    e = jnp.dot(r, w2_ref[...],
                preferred_element_type=jnp.float32) + b2_ref[...]
    el = jnp.dot(e, cw_ref[0],
                 preferred_element_type=jnp.float32) + clb_ref[0]
    el_ref[...] = _pack_bf16_halves(el)


def _edge_feats(l, edge_attr, w1, b1, w2, b2, cw, clb):
    return pl.pallas_call(
        _edge_kernel,
        grid=(E // EB,),
        in_specs=[
            pl.BlockSpec((EB, DE), lambda i: (i, 0)),
            _ew((DE, H)), _ew((1, H)),
            _ew((H, H)), _ew((1, H)),
            pl.BlockSpec((1, H, H), lambda i, _l=l: (_l, 0, 0)),
            pl.BlockSpec((1, 1, H), lambda i, _l=l: (_l, 0, 0)),
        ],
        out_specs=pl.BlockSpec((EB, H // 2), lambda i: (i, 0)),
        out_shape=jax.ShapeDtypeStruct((E, H // 2), jnp.uint32),
    )(edge_attr, w1, b1.reshape(1, H), w2, b2.reshape(1, H),
      cw, clb.reshape(L, 1, H))


NB = 1000


def _h0_kernel(x_ref, w_ref, b_ref, h_ref):
    t = (jnp.dot(x_ref[...], w_ref[...], preferred_element_type=jnp.float32)
         + b_ref[...])
    h_ref[...] = t * jax.nn.sigmoid(t)


def _h0(x_in, in_w, in_b):
    return pl.pallas_call(
        _h0_kernel,
        grid=(N // NB,),
        in_specs=[
            pl.BlockSpec((NB, DF + 1), lambda i: (i, 0)),
            _ew((DF + 1, H)), _ew((1, H)),
        ],
        out_specs=pl.BlockSpec((NB, H), lambda i: (i, 0)),
        out_shape=jax.ShapeDtypeStruct((N, H), jnp.float32),
    )(x_in, in_w, in_b.reshape(1, H))


def _sc_body(h_hbm, e_hbm, src_hbm, dst_hbm, out_hbm,
             src_v, dst_v, hrows_v, el_v, agg_sh, sem_g, sem_e, sem_i):
    cid = lax.axis_index("c")
    sid = lax.axis_index("s")
    tid = cid * NS + sid

    def zbody(i, _):
        for kk in range(H // 16):
            hrows_v[0, i, pl.ds(kk * 16, 16)] = jnp.zeros((16,), jnp.float32)
        return 0
    lax.fori_loop(0, K, zbody, 0)
    for j in range(RPS // K):
        pltpu.sync_copy(hrows_v.at[0], agg_sh.at[pl.ds(sid * RPS + j * K, K)])
    plsc.subcore_barrier()

    def issue_idx(ch, p):
        pltpu.async_copy(src_hbm.at[tid, pl.ds(ch, 1)],
                         src_v.at[pl.ds(p, 1)], sem_i.at[p])
        pltpu.async_copy(dst_hbm.at[tid, pl.ds(ch, 1)],
                         dst_v.at[pl.ds(p, 1)], sem_i.at[p])

    def wait_idx(ch, p):
        pltpu.make_async_copy(src_hbm.at[tid, pl.ds(ch, 1)],
                              src_v.at[pl.ds(p, 1)], sem_i.at[p]).wait()
        pltpu.make_async_copy(dst_hbm.at[tid, pl.ds(ch, 1)],
                              dst_v.at[pl.ds(p, 1)], sem_i.at[p]).wait()

    def issue_data(ch, p):
        pltpu.async_copy(h_hbm.at[src_v.at[p]], hrows_v.at[p], sem_g.at[p])
        pltpu.async_copy(e_hbm.at[pl.ds(tid * EPT + ch * K, K)],
                         el_v.at[p], sem_e.at[p])

    def wait_data(ch, p):
        pltpu.make_async_copy(h_hbm.at[src_v.at[p]], hrows_v.at[p],
                              sem_g.at[p]).wait()
        pltpu.make_async_copy(e_hbm.at[pl.ds(tid * EPT + ch * K, K)],
                              el_v.at[p], sem_e.at[p]).wait()

    pltpu.sync_copy(src_hbm.at[tid, pl.ds(0, 1)], src_v.at[pl.ds(0, 1)])
    pltpu.sync_copy(dst_hbm.at[tid, pl.ds(0, 1)], dst_v.at[pl.ds(0, 1)])
    issue_data(0, 0)
    issue_idx(1, 1)

    def compute_scatter(p):
        himask = jnp.full((16,), 0xFFFF0000, jnp.uint32)
        sixteen = jnp.full((16,), 16, jnp.uint32)

        def rbody(j, _):
            for kk in range(H // 32):
                sl = pl.ds(kk * 16, 16)
                slh = pl.ds(H // 2 + kk * 16, 16)
                eu = el_v[p, j, sl]
                e_lo = lax.bitcast_convert_type(eu << sixteen, jnp.float32)
                e_hi = lax.bitcast_convert_type(eu & himask, jnp.float32)
                hrows_v[p, j, sl] = jnp.maximum(
                    e_lo + hrows_v[p, j, sl], 0.0)
                hrows_v[p, j, slh] = jnp.maximum(
                    e_hi + hrows_v[p, j, slh], 0.0)
            return 0
        lax.fori_loop(0, K, rbody, 0)
        pltpu.sync_copy(hrows_v.at[p], agg_sh.at[dst_v.at[p]], add=True)

    def step(ch, p):
        wait_idx(ch + 1, 1 - p)
        issue_data(ch + 1, 1 - p)
        wait_data(ch, p)
        compute_scatter(p)
        issue_idx(ch + 2, p)

    def body(t, _):
        ch = t * 2
        step(ch, 0)
        step(ch + 1, 1)
        return 0
    lax.fori_loop(0, (NITER - 3) // 2, body, 0)

    c3, c2, c1 = (jnp.int32(NITER - 3), jnp.int32(NITER - 2),
                  jnp.int32(NITER - 1))
    wait_idx(c2, 1)
    issue_data(c2, 1)
    wait_data(c3, 0)
    compute_scatter(0)
    issue_idx(c1, 0)

    wait_idx(c1, 0)
    issue_data(c1, 0)
    wait_data(c2, 1)
    compute_scatter(1)

    wait_data(c1, 0)
    compute_scatter(0)

    plsc.subcore_barrier()
    for j in range(RPS // ZR):
        rows = pl.ds(sid * RPS + j * ZR, ZR)
        pltpu.sync_copy(agg_sh.at[rows], out_hbm.at[cid, rows])


@functools.lru_cache(maxsize=1)
def _get_sc_agg():
    mesh = plsc.VectorSubcoreMesh(core_axis_name="c", subcore_axis_name="s",
                                  num_cores=NC, num_subcores=NS)
    return pl.kernel(
        _sc_body,
        out_type=jax.ShapeDtypeStruct((NC, NPAD, H), jnp.float32),
        mesh=mesh,
        scratch_types=[
            pltpu.VMEM((2, K), jnp.int32),
            pltpu.VMEM((2, K), jnp.int32),
            pltpu.VMEM((2, K, H), jnp.float32),
            pltpu.VMEM((2, K, H // 2), jnp.uint32),
            pltpu.VMEM_SHARED((NPAD, H), jnp.float32),
            pltpu.SemaphoreType.DMA((2,)),
            pltpu.SemaphoreType.DMA((2,)),
            pltpu.SemaphoreType.DMA((2,)),
        ],
    )


def _node_kernel(h_ref, a0_ref, a1_ref, eps_ref, w1_ref, b1_ref, w2_ref,
                 b2_ref, g_ref, bb_ref, out_ref):
    z = (1.0 + eps_ref[0, 0]) * h_ref[...] + a0_ref[0] + a1_ref[0]
    t = jnp.dot(z, w1_ref[...], preferred_element_type=jnp.float32) + b1_ref[...]
    t = t * jax.nn.sigmoid(t)
    hn = jnp.dot(t, w2_ref[...], preferred_element_type=jnp.float32) + b2_ref[...]
    mu = jnp.mean(hn, axis=1, keepdims=True)
    var = jnp.mean((hn - mu) * (hn - mu), axis=1, keepdims=True)
    hn = (hn - mu) / jnp.sqrt(var + 1e-5) * g_ref[...] + bb_ref[...]
    out_ref[...] = h_ref[...] + hn * jax.nn.sigmoid(hn)


def _node_score_kernel(h_ref, a0_ref, a1_ref, eps_ref, w1_ref, b1_ref,
                       w2_ref, b2_ref, g_ref, bb_ref, aw1_ref, ab1_ref,
                       aw2_ref, ab2_ref, out_ref, s_ref):
    z = (1.0 + eps_ref[0, 0]) * h_ref[...] + a0_ref[0] + a1_ref[0]
    t = jnp.dot(z, w1_ref[...], preferred_element_type=jnp.float32) + b1_ref[...]
    t = t * jax.nn.sigmoid(t)
    hn = jnp.dot(t, w2_ref[...], preferred_element_type=jnp.float32) + b2_ref[...]
    mu = jnp.mean(hn, axis=1, keepdims=True)
    var = jnp.mean((hn - mu) * (hn - mu), axis=1, keepdims=True)
    hn = (hn - mu) / jnp.sqrt(var + 1e-5) * g_ref[...] + bb_ref[...]
    h = h_ref[...] + hn * jax.nn.sigmoid(hn)
    out_ref[...] = h
    ts = jnp.tanh(
        jnp.dot(h, aw1_ref[...], preferred_element_type=jnp.float32)
        + ab1_ref[...])
    s_ref[...] = jnp.dot(ts, aw2_ref[...],
                         preferred_element_type=jnp.float32) + ab2_ref[...]


def _node_update_score(h, agg2, eps_l, w1, b1, w2, b2, g, bb, aw1, ab1,
                       aw2, ab2):
    return pl.pallas_call(
        _node_score_kernel,
        grid=(N // NB,),
        in_specs=[
            pl.BlockSpec((NB, H), lambda i: (i, 0)),
            pl.BlockSpec((1, NB, H), lambda i: (0, i, 0)),
            pl.BlockSpec((1, NB, H), lambda i: (1, i, 0)),
            _ew((1, 1)),
            _ew((H, H)), _ew((1, H)),
            _ew((H, H)), _ew((1, H)),
            _ew((1, H)), _ew((1, H)),
            _ew((H, H)), _ew((1, H)), _ew((H, 1)), _ew((1, 1)),
        ],
        out_specs=[
            pl.BlockSpec((NB, H), lambda i: (i, 0)),
            pl.BlockSpec((NB, 1), lambda i: (i, 0)),
        ],
        out_shape=[
            jax.ShapeDtypeStruct((N, H), jnp.float32),
            jax.ShapeDtypeStruct((N, 1), jnp.float32),
        ],
    )(h, agg2, agg2, eps_l, w1, b1.reshape(1, H), w2, b2.reshape(1, H),
      g.reshape(1, H), bb.reshape(1, H), aw1, ab1.reshape(1, H), aw2,
      ab2.reshape(1, 1))


def _node_update(h, agg2, eps_l, w1, b1, w2, b2, g, bb):
    return pl.pallas_call(
        _node_kernel,
        grid=(N // NB,),
        in_specs=[
            pl.BlockSpec((NB, H), lambda i: (i, 0)),
            pl.BlockSpec((1, NB, H), lambda i: (0, i, 0)),
            pl.BlockSpec((1, NB, H), lambda i: (1, i, 0)),
            _ew((1, 1)),
            _ew((H, H)), _ew((1, H)),
            _ew((H, H)), _ew((1, H)),
            _ew((1, H)), _ew((1, H)),
        ],
        out_specs=pl.BlockSpec((NB, H), lambda i: (i, 0)),
        out_shape=jax.ShapeDtypeStruct((N, H), jnp.float32),
    )(h, agg2, agg2, eps_l, w1, b1.reshape(1, H), w2, b2.reshape(1, H),
      g.reshape(1, H), bb.reshape(1, H))


def _score_kernel(h_ref, w1_ref, b1_ref, w2_ref, b2_ref, s_ref):
    t = jnp.tanh(
        jnp.dot(h_ref[...], w1_ref[...], preferred_element_type=jnp.float32)
        + b1_ref[...])
    s_ref[...] = jnp.dot(t, w2_ref[...],
                         preferred_element_type=jnp.float32) + b2_ref[...]


def _scores(h, w1, b1, w2, b2):
    return pl.pallas_call(
        _score_kernel,
        grid=(N // NB,),
        in_specs=[
            pl.BlockSpec((NB, H), lambda i: (i, 0)),
            _ew((H, H)), _ew((1, H)), _ew((H, 1)), _ew((1, 1)),
        ],
        out_specs=pl.BlockSpec((NB, 1), lambda i: (i, 0)),
        out_shape=jax.ShapeDtypeStruct((N, 1), jnp.float32),
    )(h, w1, b1.reshape(1, H), w2, b2.reshape(1, 1))


def _pool_kernel(sall_ref, h_ref, w1_ref, b1_ref, w2_ref, b2_ref, q_ref):
    sall = sall_ref[...]
    mx = jnp.max(sall)
    u = jnp.exp(sall - mx)
    ssum = jnp.sum(u)
    segexp = jnp.sum(u, axis=1, keepdims=True)
    vs = [jnp.dot(u[bb:bb + 1], h_ref[bb],
                  preferred_element_type=jnp.float32,
                  precision=lax.Precision.HIGHEST) for bb in range(B)]
    v = jnp.concatenate(vs, axis=0)
    pooled = v / (ssum * (segexp / ssum + 1e-8))
    t = jnp.dot(pooled, w1_ref[...],
                preferred_element_type=jnp.float32) + b1_ref[...]
    t = t * jax.nn.sigmoid(t)
    q_ref[...] = jnp.dot(t, w2_ref[...],
                         preferred_element_type=jnp.float32) + b2_ref[...]


def _pool(s_r, h3, w1, b1, w2, b2):
    return pl.pallas_call(
        _pool_kernel,
        in_specs=[
            pl.BlockSpec((B, NPG), lambda: (0, 0)),
            pl.BlockSpec((B, NPG, H), lambda: (0, 0, 0)),
            pl.BlockSpec((H, H), lambda: (0, 0)),
            pl.BlockSpec((1, H), lambda: (0, 0)),
            pl.BlockSpec((H, 1), lambda: (0, 0)),
            pl.BlockSpec((1, 1), lambda: (0, 0)),
        ],
        out_specs=pl.BlockSpec((B, 1), lambda: (0, 0)),
        out_shape=jax.ShapeDtypeStruct((B, 1), jnp.float32),
    )(s_r, h3, w1, b1.reshape(1, H), w2, b2.reshape(1, 1))


def kernel(x, edge_index, edge_attr, batch, ptr, c, edge_W1, edge_b1,
           edge_W2, edge_b2, in_W, in_b, conv_lin_W, conv_lin_b, conv_W1,
           conv_b1, conv_W2, conv_b2, eps, ln_g, ln_b, attn_W1, attn_b1,
           attn_W2, attn_b2, head_W1, head_b1, head_W2, head_b2):
    e0 = _edge_feats(0, edge_attr, edge_W1, edge_b1, edge_W2, edge_b2,
                     conv_lin_W, conv_lin_b)
    x_in = jnp.concatenate([x, c.reshape(N, 1)], axis=1)
    h = _h0(x_in, in_W, in_b)
    e1 = _edge_feats(1, edge_attr, edge_W1, edge_b1, edge_W2, edge_b2,
                     conv_lin_W, conv_lin_b)

    srcr = edge_index[0].reshape(NW, NITER, K)
    dstr = edge_index[1].reshape(NW, NITER, K)
    els = (e0, e1)
    sc_agg = _get_sc_agg()
    agg2 = sc_agg(h, els[0], srcr, dstr)
    h = _node_update(h, agg2, eps[0].reshape(1, 1), conv_W1[0], conv_b1[0],
                     conv_W2[0], conv_b2[0], ln_g[0], ln_b[0])
    agg2 = sc_agg(h, els[1], srcr, dstr)
    h, s = _node_update_score(h, agg2, eps[1].reshape(1, 1), conv_W1[1],
                              conv_b1[1], conv_W2[1], conv_b2[1], ln_g[1],
                              ln_b[1], attn_W1, attn_b1, attn_W2, attn_b2)
    q = _pool(s.reshape(B, NPG), h.reshape(B, NPG, H), head_W1, head_b1,
              head_W2, head_b2)
    return q[:, 0]


# speedup vs baseline: 1.2449x; 1.0002x over previous
import functools

import jax
import jax.numpy as jnp
from jax import lax
from jax.experimental import pallas as pl
from jax.experimental.pallas import tpu as pltpu
from jax.experimental.pallas import tpu_sc as plsc

N = 10000
E = 320000
DF = 128
DE = 16
H = 128
L = 2
B = 4
NPG = N // B

NC = 2
NS = 16
NW = NC * NS
EPT = E // NW
K = 80
NITER = EPT // K
NPAD = 10240
RPS = NPAD // NS
ZR = 128

_ew = functools.partial(pl.BlockSpec, index_map=lambda i: (0, 0))


def _pack_bf16_halves(v):
    u = lax.bitcast_convert_type(v, jnp.uint32)
    r16 = (u + jnp.uint32(0x7FFF)
           + ((u >> jnp.uint32(16)) & jnp.uint32(1))) >> jnp.uint32(16)
    lo = r16[:, :H // 2]
    hi = r16[:, H // 2:]
    return lo | (hi << jnp.uint32(16))


EB = 4000


def _edge_kernel(ea_ref, w1_ref, b1_ref, w2_ref, b2_ref, cw_ref, clb_ref,
                 el_ref):
    r = jnp.maximum(
        jnp.dot(ea_ref[...], w1_ref[...], preferred_element_type=jnp.float32)
        + b1_ref[...], 0.0)
    e = jnp.dot(r, w2_ref[...],
                preferred_element_type=jnp.float32) + b2_ref[...]
    el = jnp.dot(e, cw_ref[0],
                 preferred_element_type=jnp.float32) + clb_ref[0]
    el_ref[...] = _pack_bf16_halves(el)


def _edge_feats(l, edge_attr, w1, b1, w2, b2, cw, clb):
    return pl.pallas_call(
        _edge_kernel,
        grid=(E // EB,),
        in_specs=[
            pl.BlockSpec((EB, DE), lambda i: (i, 0)),
            _ew((DE, H)), _ew((1, H)),
            _ew((H, H)), _ew((1, H)),
            pl.BlockSpec((1, H, H), lambda i, _l=l: (_l, 0, 0)),
            pl.BlockSpec((1, 1, H), lambda i, _l=l: (_l, 0, 0)),
        ],
        out_specs=pl.BlockSpec((EB, H // 2), lambda i: (i, 0)),
        out_shape=jax.ShapeDtypeStruct((E, H // 2), jnp.uint32),
    )(edge_attr, w1, b1.reshape(1, H), w2, b2.reshape(1, H),
      cw, clb.reshape(L, 1, H))


NB = 1000


def _h0_kernel(x_ref, w_ref, b_ref, h_ref):
    t = (jnp.dot(x_ref[...], w_ref[...], preferred_element_type=jnp.float32)
         + b_ref[...])
    h_ref[...] = t * jax.nn.sigmoid(t)


def _h0(x_in, in_w, in_b):
    return pl.pallas_call(
        _h0_kernel,
        grid=(N // NB,),
        in_specs=[
            pl.BlockSpec((NB, DF + 1), lambda i: (i, 0)),
            _ew((DF + 1, H)), _ew((1, H)),
        ],
        out_specs=pl.BlockSpec((NB, H), lambda i: (i, 0)),
        out_shape=jax.ShapeDtypeStruct((N, H), jnp.float32),
    )(x_in, in_w, in_b.reshape(1, H))


def _sc_body(h_hbm, e_hbm, src_hbm, dst_hbm, out_hbm,
             src_v, dst_v, hrows_v, el_v, agg_sh, sem_g, sem_e, sem_i):
    cid = lax.axis_index("c")
    sid = lax.axis_index("s")
    tid = cid * NS + sid

    def zbody(i, _):
        for kk in range(H // 16):
            hrows_v[0, i, pl.ds(kk * 16, 16)] = jnp.zeros((16,), jnp.float32)
        return 0
    lax.fori_loop(0, K, zbody, 0)
    for j in range(RPS // K):
        pltpu.sync_copy(hrows_v.at[0], agg_sh.at[pl.ds(sid * RPS + j * K, K)])
    plsc.subcore_barrier()

    def issue_idx(ch, p):
        pltpu.async_copy(src_hbm.at[tid, pl.ds(ch, 1)],
                         src_v.at[pl.ds(p, 1)], sem_i.at[p])
        pltpu.async_copy(dst_hbm.at[tid, pl.ds(ch, 1)],
                         dst_v.at[pl.ds(p, 1)], sem_i.at[p])

    def wait_idx(ch, p):
        pltpu.make_async_copy(src_hbm.at[tid, pl.ds(ch, 1)],
                              src_v.at[pl.ds(p, 1)], sem_i.at[p]).wait()
        pltpu.make_async_copy(dst_hbm.at[tid, pl.ds(ch, 1)],
                              dst_v.at[pl.ds(p, 1)], sem_i.at[p]).wait()

    def issue_data(ch, p):
        pltpu.async_copy(h_hbm.at[src_v.at[p]], hrows_v.at[p], sem_g.at[p])
        pltpu.async_copy(e_hbm.at[pl.ds(tid * EPT + ch * K, K)],
                         el_v.at[p], sem_e.at[p])

    def wait_data(ch, p):
        pltpu.make_async_copy(h_hbm.at[src_v.at[p]], hrows_v.at[p],
                              sem_g.at[p]).wait()
        pltpu.make_async_copy(e_hbm.at[pl.ds(tid * EPT + ch * K, K)],
                              el_v.at[p], sem_e.at[p]).wait()

    pltpu.sync_copy(src_hbm.at[tid, pl.ds(0, 1)], src_v.at[pl.ds(0, 1)])
    pltpu.sync_copy(dst_hbm.at[tid, pl.ds(0, 1)], dst_v.at[pl.ds(0, 1)])
    issue_data(0, 0)
    issue_idx(1, 1)

    def compute_scatter(p):
        himask = jnp.full((16,), 0xFFFF0000, jnp.uint32)
        sixteen = jnp.full((16,), 16, jnp.uint32)

        def rbody(j, _):
            for kk in range(H // 32):
                sl = pl.ds(kk * 16, 16)
                slh = pl.ds(H // 2 + kk * 16, 16)
                eu = el_v[p, j, sl]
                e_lo = lax.bitcast_convert_type(eu << sixteen, jnp.float32)
                e_hi = lax.bitcast_convert_type(eu & himask, jnp.float32)
                hrows_v[p, j, sl] = jnp.maximum(
                    e_lo + hrows_v[p, j, sl], 0.0)
                hrows_v[p, j, slh] = jnp.maximum(
                    e_hi + hrows_v[p, j, slh], 0.0)
            return 0
        lax.fori_loop(0, K, rbody, 0)
        pltpu.sync_copy(hrows_v.at[p], agg_sh.at[dst_v.at[p]], add=True)

    def step(ch, p):
        wait_idx(ch + 1, 1 - p)
        issue_data(ch + 1, 1 - p)
        wait_data(ch, p)
        compute_scatter(p)
        issue_idx(ch + 2, p)

    def body(t, _):
        ch = t * 2
        step(ch, 0)
        step(ch + 1, 1)
        return 0
    lax.fori_loop(0, (NITER - 3) // 2, body, 0)

    c3, c2, c1 = (jnp.int32(NITER - 3), jnp.int32(NITER - 2),
                  jnp.int32(NITER - 1))
    wait_idx(c2, 1)
    issue_data(c2, 1)
    wait_data(c3, 0)
    compute_scatter(0)
    issue_idx(c1, 0)

    wait_idx(c1, 0)
    issue_data(c1, 0)
    wait_data(c2, 1)
    compute_scatter(1)

    wait_data(c1, 0)
    compute_scatter(0)

    plsc.subcore_barrier()
    for j in range(RPS // ZR):
        rows = pl.ds(sid * RPS + j * ZR, ZR)
        pltpu.sync_copy(agg_sh.at[rows], out_hbm.at[cid, rows])


@functools.lru_cache(maxsize=1)
def _get_sc_agg():
    mesh = plsc.VectorSubcoreMesh(core_axis_name="c", subcore_axis_name="s",
                                  num_cores=NC, num_subcores=NS)
    return pl.kernel(
        _sc_body,
        out_type=jax.ShapeDtypeStruct((NC, NPAD, H), jnp.float32),
        mesh=mesh,
        scratch_types=[
            pltpu.VMEM((2, K), jnp.int32),
            pltpu.VMEM((2, K), jnp.int32),
            pltpu.VMEM((2, K, H), jnp.float32),
            pltpu.VMEM((2, K, H // 2), jnp.uint32),
            pltpu.VMEM_SHARED((NPAD, H), jnp.float32),
            pltpu.SemaphoreType.DMA((2,)),
            pltpu.SemaphoreType.DMA((2,)),
            pltpu.SemaphoreType.DMA((2,)),
        ],
    )


def _node_kernel(h_ref, a0_ref, a1_ref, eps_ref, w1_ref, b1_ref, w2_ref,
                 b2_ref, g_ref, bb_ref, out_ref):
    z = (1.0 + eps_ref[0, 0]) * h_ref[...] + a0_ref[0] + a1_ref[0]
    t = jnp.dot(z, w1_ref[...], preferred_element_type=jnp.float32) + b1_ref[...]
    t = t * jax.nn.sigmoid(t)
    hn = jnp.dot(t, w2_ref[...], preferred_element_type=jnp.float32) + b2_ref[...]
    mu = jnp.mean(hn, axis=1, keepdims=True)
    var = jnp.mean((hn - mu) * (hn - mu), axis=1, keepdims=True)
    hn = (hn - mu) / jnp.sqrt(var + 1e-5) * g_ref[...] + bb_ref[...]
    out_ref[...] = h_ref[...] + hn * jax.nn.sigmoid(hn)


def _node_score_kernel(h_ref, a0_ref, a1_ref, eps_ref, w1_ref, b1_ref,
                       w2_ref, b2_ref, g_ref, bb_ref, aw1_ref, ab1_ref,
                       aw2_ref, ab2_ref, out_ref, s_ref):
    z = (1.0 + eps_ref[0, 0]) * h_ref[...] + a0_ref[0] + a1_ref[0]
    t = jnp.dot(z, w1_ref[...], preferred_element_type=jnp.float32) + b1_ref[...]
    t = t * jax.nn.sigmoid(t)
    hn = jnp.dot(t, w2_ref[...], preferred_element_type=jnp.float32) + b2_ref[...]
    mu = jnp.mean(hn, axis=1, keepdims=True)
    var = jnp.mean((hn - mu) * (hn - mu), axis=1, keepdims=True)
    hn = (hn - mu) / jnp.sqrt(var + 1e-5) * g_ref[...] + bb_ref[...]
    h = h_ref[...] + hn * jax.nn.sigmoid(hn)
    out_ref[...] = h
    ts = jnp.tanh(
        jnp.dot(h, aw1_ref[...], preferred_element_type=jnp.float32)
        + ab1_ref[...])
    s_ref[...] = jnp.dot(ts, aw2_ref[...],
                         preferred_element_type=jnp.float32) + ab2_ref[...]


def _node_update_score(h, agg2, eps_l, w1, b1, w2, b2, g, bb, aw1, ab1,
                       aw2, ab2):
    return pl.pallas_call(
        _node_score_kernel,
        grid=(N // NB,),
        in_specs=[
            pl.BlockSpec((NB, H), lambda i: (i, 0)),
            pl.BlockSpec((1, NB, H), lambda i: (0, i, 0)),
            pl.BlockSpec((1, NB, H), lambda i: (1, i, 0)),
            _ew((1, 1)),
            _ew((H, H)), _ew((1, H)),
            _ew((H, H)), _ew((1, H)),
            _ew((1, H)), _ew((1, H)),
            _ew((H, H)), _ew((1, H)), _ew((H, 1)), _ew((1, 1)),
        ],
        out_specs=[
            pl.BlockSpec((NB, H), lambda i: (i, 0)),
            pl.BlockSpec((NB, 1), lambda i: (i, 0)),
        ],
        out_shape=[
            jax.ShapeDtypeStruct((N, H), jnp.float32),
            jax.ShapeDtypeStruct((N, 1), jnp.float32),
        ],
    )(h, agg2, agg2, eps_l, w1, b1.reshape(1, H), w2, b2.reshape(1, H),
      g.reshape(1, H), bb.reshape(1, H), aw1, ab1.reshape(1, H), aw2,
      ab2.reshape(1, 1))


def _node_update(h, agg2, eps_l, w1, b1, w2, b2, g, bb):
    return pl.pallas_call(
        _node_kernel,
        grid=(N // NB,),
        in_specs=[
            pl.BlockSpec((NB, H), lambda i: (i, 0)),
            pl.BlockSpec((1, NB, H), lambda i: (0, i, 0)),
            pl.BlockSpec((1, NB, H), lambda i: (1, i, 0)),
            _ew((1, 1)),
            _ew((H, H)), _ew((1, H)),
            _ew((H, H)), _ew((1, H)),
            _ew((1, H)), _ew((1, H)),
        ],
        out_specs=pl.BlockSpec((NB, H), lambda i: (i, 0)),
        out_shape=jax.ShapeDtypeStruct((N, H), jnp.float32),
    )(h, agg2, agg2, eps_l, w1, b1.reshape(1, H), w2, b2.reshape(1, H),
      g.reshape(1, H), bb.reshape(1, H))


def _pool_kernel(sall_ref, h_ref, w1_ref, b1_ref, w2_ref, b2_ref, q_ref):
    sall = sall_ref[...]
    mx = jnp.max(sall)
    u = jnp.exp(sall - mx)
    ssum = jnp.sum(u)
    segexp = jnp.sum(u, axis=1, keepdims=True)
    vs = [jnp.dot(u[bb:bb + 1], h_ref[bb],
                  preferred_element_type=jnp.float32,
                  precision=lax.Precision.HIGHEST) for bb in range(B)]
    v = jnp.concatenate(vs, axis=0)
    pooled = v / (ssum * (segexp / ssum + 1e-8))
    t = jnp.dot(pooled, w1_ref[...],
                preferred_element_type=jnp.float32) + b1_ref[...]
    t = t * jax.nn.sigmoid(t)
    q_ref[...] = jnp.dot(t, w2_ref[...],
                         preferred_element_type=jnp.float32) + b2_ref[...]


def _pool(s_r, h3, w1, b1, w2, b2):
    return pl.pallas_call(
        _pool_kernel,
        in_specs=[
            pl.BlockSpec((B, NPG), lambda: (0, 0)),
            pl.BlockSpec((B, NPG, H), lambda: (0, 0, 0)),
            pl.BlockSpec((H, H), lambda: (0, 0)),
            pl.BlockSpec((1, H), lambda: (0, 0)),
            pl.BlockSpec((H, 1), lambda: (0, 0)),
            pl.BlockSpec((1, 1), lambda: (0, 0)),
        ],
        out_specs=pl.BlockSpec((B, 1), lambda: (0, 0)),
        out_shape=jax.ShapeDtypeStruct((B, 1), jnp.float32),
    )(s_r, h3, w1, b1.reshape(1, H), w2, b2.reshape(1, 1))


def kernel(x, edge_index, edge_attr, batch, ptr, c, edge_W1, edge_b1,
           edge_W2, edge_b2, in_W, in_b, conv_lin_W, conv_lin_b, conv_W1,
           conv_b1, conv_W2, conv_b2, eps, ln_g, ln_b, attn_W1, attn_b1,
           attn_W2, attn_b2, head_W1, head_b1, head_W2, head_b2):
    e0 = _edge_feats(0, edge_attr, edge_W1, edge_b1, edge_W2, edge_b2,
                     conv_lin_W, conv_lin_b)
    x_in = jnp.concatenate([x, c.reshape(N, 1)], axis=1)
    h = _h0(x_in, in_W, in_b)
    e1 = _edge_feats(1, edge_attr, edge_W1, edge_b1, edge_W2, edge_b2,
                     conv_lin_W, conv_lin_b)

    srcr = edge_index[0].reshape(NW, NITER, K)
    dstr = edge_index[1].reshape(NW, NITER, K)
    els = (e0, e1)
    sc_agg = _get_sc_agg()
    agg2 = sc_agg(h, els[0], srcr, dstr)
    h = _node_update(h, agg2, eps[0].reshape(1, 1), conv_W1[0], conv_b1[0],
                     conv_W2[0], conv_b2[0], ln_g[0], ln_b[0])
    agg2 = sc_agg(h, els[1], srcr, dstr)
    h, s = _node_update_score(h, agg2, eps[1].reshape(1, 1), conv_W1[1],
                              conv_b1[1], conv_W2[1], conv_b2[1], ln_g[1],
                              ln_b[1], attn_W1, attn_b1, attn_W2, attn_b2)
    q = _pool(s.reshape(B, NPG), h.reshape(B, NPG, H), head_W1, head_b1,
              head_W2, head_b2)
    return q[:, 0]
